# baseline (speedup 1.0000x reference)
import jax
import jax.numpy as jnp
from jax.experimental import pallas as pl
from jax.experimental.pallas import tpu as pltpu

N = 10000
NFEAT = 128
NHID = 128
N_CLUSTERS = 10
ALPHA = 0.2

BM = 400


def _fused_kernel(x_ref, w_ref, adj_ref, adj2_ref, b_ref, mu_ref, h_ref, q_ref, s_ref):
    @pl.when(pl.program_id(0) == 0)
    def _compute_support():
        s_ref[...] = jax.lax.dot_general(
            x_ref[...], w_ref[...],
            (((1,), (0,)), ((), ())),
            preferred_element_type=jnp.float32,
        )

    sup = s_ref[...]
    h1 = jax.lax.dot_general(
        adj_ref[...], sup,
        (((1,), (0,)), ((), ())),
        preferred_element_type=jnp.float32,
    )
    h2r = jax.lax.dot_general(
        adj2_ref[...], sup,
        (((1,), (0,)), ((), ())),
        preferred_element_type=jnp.float32,
    )
    h = jnp.concatenate([h1, h2r], axis=0) + b_ref[...]
    h_ref[...] = h

    mu = mu_ref[...]
    hmu = jax.lax.dot_general(
        h, mu, (((1,), (1,)), ((), ())),
        preferred_element_type=jnp.float32,
    )
    h2 = jnp.sum(h * h, axis=1, keepdims=True)
    mu2 = jax.lax.dot_general(
        jnp.ones((1, NHID), jnp.float32), mu * mu,
        (((1,), (1,)), ((), ())),
        preferred_element_type=jnp.float32,
    )
    d2 = h2 + mu2 - 2.0 * hmu
    t = 1.0 / (1.0 + d2 / ALPHA + 1e-8)
    q = jnp.exp((ALPHA + 1.0) * jnp.log(t))
    q_ref[...] = q / jnp.sum(q, axis=1, keepdims=True)


def kernel(x, adj, W, b, mu):
    b2 = b.reshape(1, NHID)
    grid = (N // BM,)
    h, q = pl.pallas_call(
        _fused_kernel,
        grid=grid,
        in_specs=[
            pl.BlockSpec((N, NFEAT), lambda i: (0, 0)),
            pl.BlockSpec((NFEAT, NHID), lambda i: (0, 0)),
            pl.BlockSpec((BM // 2, N), lambda i: (2 * i, 0)),
            pl.BlockSpec((BM // 2, N), lambda i: (2 * i + 1, 0)),
            pl.BlockSpec((1, NHID), lambda i: (0, 0)),
            pl.BlockSpec((N_CLUSTERS, NHID), lambda i: (0, 0)),
        ],
        out_specs=[
            pl.BlockSpec((BM, NHID), lambda i: (i, 0)),
            pl.BlockSpec((BM, N_CLUSTERS), lambda i: (i, 0)),
        ],
        out_shape=[
            jax.ShapeDtypeStruct((N, NHID), jnp.float32),
            jax.ShapeDtypeStruct((N, N_CLUSTERS), jnp.float32),
        ],
        scratch_shapes=[pltpu.VMEM((N, NHID), jnp.float32)],
    )(x, W, adj, adj, b2, mu)
    return (h, q)


# speedup vs baseline: 1.0134x; 1.0134x over previous
import jax
import jax.numpy as jnp
from jax.experimental import pallas as pl
from jax.experimental.pallas import tpu as pltpu

N = 10000
NFEAT = 128
NHID = 128
N_CLUSTERS = 10
ALPHA = 0.2

BM = 400


def _fused_kernel(x_ref, w_ref, adj_ref, b_ref, mu_ref, h_ref, q_ref, s_ref):
    @pl.when(pl.program_id(0) == 0)
    def _compute_support():
        s_ref[...] = jax.lax.dot_general(
            x_ref[...], w_ref[...],
            (((1,), (0,)), ((), ())),
            preferred_element_type=jnp.float32,
        )

    h = jax.lax.dot_general(
        adj_ref[...], s_ref[...],
        (((1,), (0,)), ((), ())),
        preferred_element_type=jnp.float32,
    )
    h = h + b_ref[...]
    h_ref[...] = h

    mu = mu_ref[...]
    hmu = jax.lax.dot_general(
        h, mu, (((1,), (1,)), ((), ())),
        preferred_element_type=jnp.float32,
    )
    h2 = jnp.sum(h * h, axis=1, keepdims=True)
    mu2 = jax.lax.dot_general(
        jnp.ones((1, NHID), jnp.float32), mu * mu,
        (((1,), (1,)), ((), ())),
        preferred_element_type=jnp.float32,
    )
    d2 = h2 + mu2 - 2.0 * hmu
    t = 1.0 / (1.0 + d2 / ALPHA + 1e-8)
    q = jnp.exp((ALPHA + 1.0) * jnp.log(t))
    q_ref[...] = q / jnp.sum(q, axis=1, keepdims=True)


def kernel(x, adj, W, b, mu):
    b2 = b.reshape(1, NHID)
    grid = (N // BM,)
    h, q = pl.pallas_call(
        _fused_kernel,
        grid=grid,
        in_specs=[
            pl.BlockSpec((N, NFEAT), lambda i: (0, 0)),
            pl.BlockSpec((NFEAT, NHID), lambda i: (0, 0)),
            pl.BlockSpec((BM, N), lambda i: (i, 0)),
            pl.BlockSpec((1, NHID), lambda i: (0, 0)),
            pl.BlockSpec((N_CLUSTERS, NHID), lambda i: (0, 0)),
        ],
        out_specs=[
            pl.BlockSpec((BM, NHID), lambda i: (i, 0)),
            pl.BlockSpec((BM, N_CLUSTERS), lambda i: (i, 0)),
        ],
        out_shape=[
            jax.ShapeDtypeStruct((N, NHID), jnp.float32),
            jax.ShapeDtypeStruct((N, N_CLUSTERS), jnp.float32),
        ],
        scratch_shapes=[pltpu.VMEM((N, NHID), jnp.float32)],
    )(x, W, adj, b2, mu)
    return (h, q)
